# baseline (speedup 1.0000x reference)
import jax
import jax.numpy as jnp
from jax import lax
from jax.experimental import pallas as pl
from jax.experimental.pallas import tpu as pltpu
from jax.experimental.pallas import tpu_sc as plsc

BATCH = 1024
TAU = 30.0
NCLS = 10
NC, NS, L = 2, 16, 16
NW = NC * NS
OUT_PAD = 8192
BPW = OUT_PAD // NW
GRP = 16
NGRP = BPW // GRP
RQ = 8

_GATE_K = (
    (0, 0, 0, 0), (0, 0, 0, 1), (0, 1, 0, -1), (0, 1, 0, 0),
    (0, 0, 1, -1), (0, 0, 1, 0), (0, 1, 1, -2), (0, 1, 1, -1),
    (1, -1, -1, 1), (1, -1, -1, 2), (1, 0, -1, 0), (1, 0, -1, 1),
    (1, -1, 0, 0), (1, -1, 0, 1), (1, 0, 0, -1), (1, 0, 0, 0),
)


def _coef_tc(wall):
    def body(w_ref, k_ref, o_ref):
        w = w_ref[...]
        m = jnp.max(w, axis=-1, keepdims=True)
        e = jnp.exp(w - m)
        p = e / jnp.sum(e, axis=-1, keepdims=True)
        o_ref[...] = jax.lax.dot(p, k_ref[...], precision=lax.Precision.HIGHEST)

    n = wall.shape[0]
    blk = 2048
    return pl.pallas_call(
        body,
        grid=(n // blk,),
        in_specs=[
            pl.BlockSpec((blk, 16), lambda i: (i, 0)),
            pl.BlockSpec((16, 4), lambda i: (0, 0)),
        ],
        out_specs=pl.BlockSpec((blk, 4), lambda i: (i, 0)),
        out_shape=jax.ShapeDtypeStruct((n, 4), jnp.float32),
    )(wall, jnp.asarray(_GATE_K, dtype=jnp.float32))


def _sc_layer(table, aidx, bidx, cfs, nb):
    mesh = plsc.VectorSubcoreMesh(core_axis_name="c", subcore_axis_name="s")

    def body(tab, ai, bi, cf, out, aiv, biv, cfv,
             abufs, bbufs, obufs, sems_a, sems_b, sems_o):
        wid = lax.axis_index("s") * NC + lax.axis_index("c")
        base = wid * BPW
        pltpu.sync_copy(ai.at[pl.ds(base, BPW)], aiv)
        pltpu.sync_copy(bi.at[pl.ds(base, BPW)], biv)
        pltpu.sync_copy(cf.at[:, pl.ds(base, BPW)], cfv)

        def issue(g):
            s = g % 2
            r0 = g * GRP
            cpa = pltpu.async_copy(
                tab.at[aiv.at[pl.ds(r0, GRP)]], abufs[s], sems_a[s])
            cpb = pltpu.async_copy(
                tab.at[biv.at[pl.ds(r0, GRP)]], bbufs[s], sems_b[s])
            return cpa, cpb

        pend = {0: issue(0)}
        out_pend = {}
        for g in range(NGRP):
            s = g % 2
            if g + 1 < NGRP:
                pend[g + 1] = issue(g + 1)
            cpa, cpb = pend.pop(g)
            cpa.wait()
            cpb.wait()
            if g >= 2:
                out_pend.pop(g - 2).wait()
            abuf, bbuf, obuf = abufs[s], bbufs[s], obufs[s]
            r0 = g * GRP
            c0v = cfv[0, pl.ds(r0, GRP)]
            c1v = cfv[1, pl.ds(r0, GRP)]
            c2v = cfv[2, pl.ds(r0, GRP)]
            c3v = cfv[3, pl.ds(r0, GRP)]
            for q in range(GRP // RQ):
                rows = [q * RQ + i for i in range(RQ)]
                cs = [(c0v[r], c1v[r], c2v[r], c3v[r]) for r in rows]

                def col_fn(j, carry2, rows=rows, cs=cs,
                           abuf=abuf, bbuf=bbuf, obuf=obuf):
                    sl = pl.ds(j * L, L)
                    for r, (c0, c1, c2, c3) in zip(rows, cs):
                        av = abuf[r, sl]
                        bv = bbuf[r, sl]
                        obuf[r, sl] = (c0 + c1 * av) + (c2 + c3 * av) * bv
                    return carry2

                lax.fori_loop(0, nb // L, col_fn, 0)
            out_pend[g] = pltpu.async_copy(
                obuf, out.at[pl.ds(base + r0, GRP)], sems_o[s])
        for g in sorted(out_pend):
            out_pend.pop(g).wait()

    kfn = pl.kernel(
        body,
        out_type=jax.ShapeDtypeStruct((OUT_PAD, nb), jnp.float32),
        mesh=mesh,
        scratch_types=[
            pltpu.VMEM((BPW,), jnp.int32),
            pltpu.VMEM((BPW,), jnp.int32),
            pltpu.VMEM((4, BPW), jnp.float32),
            [pltpu.VMEM((GRP, nb), jnp.float32)] * 2,
            [pltpu.VMEM((GRP, nb), jnp.float32)] * 2,
            [pltpu.VMEM((GRP, nb), jnp.float32)] * 2,
            [pltpu.SemaphoreType.DMA] * 2,
            [pltpu.SemaphoreType.DMA] * 2,
            [pltpu.SemaphoreType.DMA] * 2,
        ],
    )
    return kfn(table, aidx, bidx, cfs)


def _sc_layer3_gsum(table, aidx, bidx, cfs, nb):
    mesh = plsc.VectorSubcoreMesh(core_axis_name="c", subcore_axis_name="s")

    def body(tab, ai, bi, cf, out, aiv, biv, cfv, acc,
             abufs, bbufs, sems_a, sems_b):
        wid = lax.axis_index("s") * NC + lax.axis_index("c")
        base = wid * BPW
        pltpu.sync_copy(ai.at[pl.ds(base, BPW)], aiv)
        pltpu.sync_copy(bi.at[pl.ds(base, BPW)], biv)
        pltpu.sync_copy(cf.at[:, pl.ds(base, BPW)], cfv)

        def issue(g):
            s = g % 2
            r0 = g * GRP
            cpa = pltpu.async_copy(
                tab.at[aiv.at[pl.ds(r0, GRP)]], abufs[s], sems_a[s])
            cpb = pltpu.async_copy(
                tab.at[biv.at[pl.ds(r0, GRP)]], bbufs[s], sems_b[s])
            return cpa, cpb

        pend = {0: issue(0)}
        for g in range(NGRP):
            s = g % 2
            if g + 1 < NGRP:
                pend[g + 1] = issue(g + 1)
            cpa, cpb = pend.pop(g)
            cpa.wait()
            cpb.wait()
            abuf, bbuf = abufs[s], bbufs[s]
            r0 = g * GRP
            c0v = cfv[0, pl.ds(r0, GRP)]
            c1v = cfv[1, pl.ds(r0, GRP)]
            c2v = cfv[2, pl.ds(r0, GRP)]
            c3v = cfv[3, pl.ds(r0, GRP)]
            m1v = cfv[4, pl.ds(r0, GRP)]
            for q in range(GRP // RQ):
                rows = [q * RQ + i for i in range(RQ)]
                cs = [(c0v[r], c1v[r], c2v[r], c3v[r], m1v[r]) for r in rows]
                init = (g == 0 and q == 0)

                def col_fn(j, carry2, rows=rows, cs=cs, init=init,
                           abuf=abuf, bbuf=bbuf):
                    sl = pl.ds(j * L, L)
                    if init:
                        r0_, (c0, c1, c2, c3, m1) = rows[0], cs[0]
                        av = abuf[r0_, sl]
                        bv = bbuf[r0_, sl]
                        val = (c0 + c1 * av) + (c2 + c3 * av) * bv
                        sacc = val
                        tacc = m1 * val
                        rest = list(zip(rows[1:], cs[1:]))
                    else:
                        sacc = acc[0, sl]
                        tacc = acc[1, sl]
                        rest = list(zip(rows, cs))
                    for r, (c0, c1, c2, c3, m1) in rest:
                        av = abuf[r, sl]
                        bv = bbuf[r, sl]
                        val = (c0 + c1 * av) + (c2 + c3 * av) * bv
                        sacc = sacc + val
                        tacc = tacc + m1 * val
                    acc[0, sl] = sacc
                    acc[1, sl] = tacc
                    return carry2

                lax.fori_loop(0, nb // L, col_fn, 0)
        pltpu.sync_copy(acc, out.at[pl.ds(wid * 2, 2)])

    kfn = pl.kernel(
        body,
        out_type=jax.ShapeDtypeStruct((2 * NW, nb), jnp.float32),
        mesh=mesh,
        scratch_types=[
            pltpu.VMEM((BPW,), jnp.int32),
            pltpu.VMEM((BPW,), jnp.int32),
            pltpu.VMEM((5, BPW), jnp.float32),
            pltpu.VMEM((2, nb), jnp.float32),
            [pltpu.VMEM((GRP, nb), jnp.float32)] * 2,
            [pltpu.VMEM((GRP, nb), jnp.float32)] * 2,
            [pltpu.SemaphoreType.DMA] * 2,
            [pltpu.SemaphoreType.DMA] * 2,
        ],
    )
    return kfn(table, aidx, bidx, cfs)


def _combine_tc(partials, sel):
    def body(s_ref, p_ref, o_ref):
        o_ref[...] = jax.lax.dot(
            s_ref[...], p_ref[...], precision=lax.Precision.HIGHEST) / TAU

    return pl.pallas_call(
        body,
        out_shape=jax.ShapeDtypeStruct((NCLS, BATCH), jnp.float32),
    )(sel, partials)


def kernel(x, w1, w2, w3, a1, b1, a2, b2, a3, b3):
    xt = x.T
    n3 = w3.shape[0]
    w3p = jnp.concatenate([w3, jnp.zeros((OUT_PAD - n3, 16), jnp.float32)], 0)
    wall = jnp.concatenate([w1, w2, w3p], axis=0)
    coefs = _coef_tc(wall).T
    cf1 = coefs[:, :OUT_PAD]
    cf2 = coefs[:, OUT_PAD:2 * OUT_PAD]
    cf3 = coefs[:, 2 * OUT_PAD:]
    padi = jnp.arange(OUT_PAD - n3, dtype=jnp.int32)
    a3p = jnp.concatenate([a3, padi])
    b3p = jnp.concatenate([b3, padi])

    gsz = n3 // NCLS
    g = jnp.arange(OUT_PAD)
    c0w = (g // BPW) * BPW // gsz
    m1 = ((g // gsz == c0w + 1) & (c0w < NCLS - 1)).astype(jnp.float32)
    cf3m = jnp.concatenate(
        [cf3 * (g < n3).astype(jnp.float32)[None, :], m1[None, :]], axis=0)
    srows = [[0.0] * (2 * NW) for _ in range(NCLS)]
    for w in range(NW):
        c0 = w * BPW // gsz
        srows[c0][2 * w] += 1.0
        srows[c0][2 * w + 1] -= 1.0
        if c0 + 1 < NCLS:
            srows[c0 + 1][2 * w + 1] += 1.0
    sel = jnp.asarray(srows, dtype=jnp.float32)

    h = BATCH // 2
    xta, xtb = xt[:, :h], xt[:, h:]
    y1a = _sc_layer(xta, a1, b1, cf1, h)
    y1b = _sc_layer(xtb, a1, b1, cf1, h)
    y2a = _sc_layer(y1a, a2, b2, cf2, h)
    y2b = _sc_layer(y1b, a2, b2, cf2, h)
    pa = _sc_layer3_gsum(y2a, a3p, b3p, cf3m, h)
    pb = _sc_layer3_gsum(y2b, a3p, b3p, cf3m, h)
    cls = _combine_tc(jnp.concatenate([pa, pb], axis=1), sel)
    return cls.T


# speedup vs baseline: 1.4283x; 1.4283x over previous
import jax
import jax.numpy as jnp
from jax import lax
from jax.experimental import pallas as pl
from jax.experimental.pallas import tpu as pltpu
from jax.experimental.pallas import tpu_sc as plsc

BATCH = 1024
TAU = 30.0
NCLS = 10
NC, NS, L = 2, 16, 16
NW = NC * NS
OUT_PAD = 8192
BPW = OUT_PAD // NW
GRP = 16
NGRP = BPW // GRP
RQ = 8

_GATE_K = (
    (0, 0, 0, 0), (0, 0, 0, 1), (0, 1, 0, -1), (0, 1, 0, 0),
    (0, 0, 1, -1), (0, 0, 1, 0), (0, 1, 1, -2), (0, 1, 1, -1),
    (1, -1, -1, 1), (1, -1, -1, 2), (1, 0, -1, 0), (1, 0, -1, 1),
    (1, -1, 0, 0), (1, -1, 0, 1), (1, 0, 0, -1), (1, 0, 0, 0),
)


def _coef_tc(wall):
    def body(w_ref, k_ref, o_ref):
        w = w_ref[...]
        m = jnp.max(w, axis=-1, keepdims=True)
        e = jnp.exp(w - m)
        p = e / jnp.sum(e, axis=-1, keepdims=True)
        o_ref[...] = jax.lax.dot(p, k_ref[...], precision=lax.Precision.HIGHEST)

    n = wall.shape[0]
    blk = 2048
    return pl.pallas_call(
        body,
        grid=(n // blk,),
        in_specs=[
            pl.BlockSpec((blk, 16), lambda i: (i, 0)),
            pl.BlockSpec((16, 4), lambda i: (0, 0)),
        ],
        out_specs=pl.BlockSpec((blk, 4), lambda i: (i, 0)),
        out_shape=jax.ShapeDtypeStruct((n, 4), jnp.float32),
    )(wall, jnp.asarray(_GATE_K, dtype=jnp.float32))


def _sc_layer(table, aidx, bidx, cfs):
    mesh = plsc.VectorSubcoreMesh(core_axis_name="c", subcore_axis_name="s")

    def body(tab, ai, bi, cf, out, aiv, biv, cfv,
             abufs, bbufs, obufs, sems_a, sems_b, sems_o):
        wid = lax.axis_index("s") * NC + lax.axis_index("c")
        base = wid * BPW
        pltpu.sync_copy(ai.at[pl.ds(base, BPW)], aiv)
        pltpu.sync_copy(bi.at[pl.ds(base, BPW)], biv)
        pltpu.sync_copy(cf.at[:, pl.ds(base, BPW)], cfv)

        def issue(g):
            s = g % 2
            r0 = g * GRP
            cpa = pltpu.async_copy(
                tab.at[aiv.at[pl.ds(r0, GRP)]], abufs[s], sems_a[s])
            cpb = pltpu.async_copy(
                tab.at[biv.at[pl.ds(r0, GRP)]], bbufs[s], sems_b[s])
            return cpa, cpb

        pend = {0: issue(0)}
        out_pend = {}
        for g in range(NGRP):
            s = g % 2
            if g + 1 < NGRP:
                pend[g + 1] = issue(g + 1)
            cpa, cpb = pend.pop(g)
            cpa.wait()
            cpb.wait()
            if g >= 2:
                out_pend.pop(g - 2).wait()
            abuf, bbuf, obuf = abufs[s], bbufs[s], obufs[s]
            r0 = g * GRP
            c0v = cfv[0, pl.ds(r0, GRP)]
            c1v = cfv[1, pl.ds(r0, GRP)]
            c2v = cfv[2, pl.ds(r0, GRP)]
            c3v = cfv[3, pl.ds(r0, GRP)]
            for q in range(GRP // RQ):
                rows = [q * RQ + i for i in range(RQ)]
                cs = [(c0v[r], c1v[r], c2v[r], c3v[r]) for r in rows]

                def col_fn(j, carry2, rows=rows, cs=cs,
                           abuf=abuf, bbuf=bbuf, obuf=obuf):
                    sl = pl.ds(j * L, L)
                    for r, (c0, c1, c2, c3) in zip(rows, cs):
                        av = abuf[r, sl]
                        bv = bbuf[r, sl]
                        obuf[r, sl] = (c0 + c1 * av) + (c2 + c3 * av) * bv
                    return carry2

                lax.fori_loop(0, BATCH // L, col_fn, 0)
            out_pend[g] = pltpu.async_copy(
                obuf, out.at[pl.ds(base + r0, GRP)], sems_o[s])
        for g in sorted(out_pend):
            out_pend.pop(g).wait()

    kfn = pl.kernel(
        body,
        out_type=jax.ShapeDtypeStruct((OUT_PAD, BATCH), jnp.float32),
        mesh=mesh,
        scratch_types=[
            pltpu.VMEM((BPW,), jnp.int32),
            pltpu.VMEM((BPW,), jnp.int32),
            pltpu.VMEM((4, BPW), jnp.float32),
            [pltpu.VMEM((GRP, BATCH), jnp.float32)] * 2,
            [pltpu.VMEM((GRP, BATCH), jnp.float32)] * 2,
            [pltpu.VMEM((GRP, BATCH), jnp.float32)] * 2,
            [pltpu.SemaphoreType.DMA] * 2,
            [pltpu.SemaphoreType.DMA] * 2,
            [pltpu.SemaphoreType.DMA] * 2,
        ],
    )
    return kfn(table, aidx, bidx, cfs)


def _sc_layer3_gsum(table, aidx, bidx, cfs):
    mesh = plsc.VectorSubcoreMesh(core_axis_name="c", subcore_axis_name="s")

    def body(tab, ai, bi, cf, out, aiv, biv, cfv, acc,
             abufs, bbufs, sems_a, sems_b):
        wid = lax.axis_index("s") * NC + lax.axis_index("c")
        base = wid * BPW
        pltpu.sync_copy(ai.at[pl.ds(base, BPW)], aiv)
        pltpu.sync_copy(bi.at[pl.ds(base, BPW)], biv)
        pltpu.sync_copy(cf.at[:, pl.ds(base, BPW)], cfv)

        def issue(g):
            s = g % 2
            r0 = g * GRP
            cpa = pltpu.async_copy(
                tab.at[aiv.at[pl.ds(r0, GRP)]], abufs[s], sems_a[s])
            cpb = pltpu.async_copy(
                tab.at[biv.at[pl.ds(r0, GRP)]], bbufs[s], sems_b[s])
            return cpa, cpb

        pend = {0: issue(0)}
        for g in range(NGRP):
            s = g % 2
            if g + 1 < NGRP:
                pend[g + 1] = issue(g + 1)
            cpa, cpb = pend.pop(g)
            cpa.wait()
            cpb.wait()
            abuf, bbuf = abufs[s], bbufs[s]
            r0 = g * GRP
            c0v = cfv[0, pl.ds(r0, GRP)]
            c1v = cfv[1, pl.ds(r0, GRP)]
            c2v = cfv[2, pl.ds(r0, GRP)]
            c3v = cfv[3, pl.ds(r0, GRP)]
            m1v = cfv[4, pl.ds(r0, GRP)]
            for q in range(GRP // RQ):
                rows = [q * RQ + i for i in range(RQ)]
                cs = [(c0v[r], c1v[r], c2v[r], c3v[r], m1v[r]) for r in rows]
                init = (g == 0 and q == 0)

                def col_fn(j, carry2, rows=rows, cs=cs, init=init,
                           abuf=abuf, bbuf=bbuf):
                    sl = pl.ds(j * L, L)
                    if init:
                        r0_, (c0, c1, c2, c3, m1) = rows[0], cs[0]
                        av = abuf[r0_, sl]
                        bv = bbuf[r0_, sl]
                        val = (c0 + c1 * av) + (c2 + c3 * av) * bv
                        sacc = val
                        tacc = m1 * val
                        rest = list(zip(rows[1:], cs[1:]))
                    else:
                        sacc = acc[0, sl]
                        tacc = acc[1, sl]
                        rest = list(zip(rows, cs))
                    for r, (c0, c1, c2, c3, m1) in rest:
                        av = abuf[r, sl]
                        bv = bbuf[r, sl]
                        val = (c0 + c1 * av) + (c2 + c3 * av) * bv
                        sacc = sacc + val
                        tacc = tacc + m1 * val
                    acc[0, sl] = sacc
                    acc[1, sl] = tacc
                    return carry2

                lax.fori_loop(0, BATCH // L, col_fn, 0)
        pltpu.sync_copy(acc, out.at[pl.ds(wid * 2, 2)])

    kfn = pl.kernel(
        body,
        out_type=jax.ShapeDtypeStruct((2 * NW, BATCH), jnp.float32),
        mesh=mesh,
        scratch_types=[
            pltpu.VMEM((BPW,), jnp.int32),
            pltpu.VMEM((BPW,), jnp.int32),
            pltpu.VMEM((5, BPW), jnp.float32),
            pltpu.VMEM((2, BATCH), jnp.float32),
            [pltpu.VMEM((GRP, BATCH), jnp.float32)] * 2,
            [pltpu.VMEM((GRP, BATCH), jnp.float32)] * 2,
            [pltpu.SemaphoreType.DMA] * 2,
            [pltpu.SemaphoreType.DMA] * 2,
        ],
    )
    return kfn(table, aidx, bidx, cfs)


def _combine_tc(partials, sel):
    def body(s_ref, p_ref, o_ref):
        o_ref[...] = jax.lax.dot(
            s_ref[...], p_ref[...], precision=lax.Precision.HIGHEST) / TAU

    return pl.pallas_call(
        body,
        out_shape=jax.ShapeDtypeStruct((NCLS, BATCH), jnp.float32),
    )(sel, partials)


def kernel(x, w1, w2, w3, a1, b1, a2, b2, a3, b3):
    xt = x.T
    n3 = w3.shape[0]
    w3p = jnp.concatenate([w3, jnp.zeros((OUT_PAD - n3, 16), jnp.float32)], 0)
    wall = jnp.concatenate([w1, w2, w3p], axis=0)
    coefs = _coef_tc(wall).T
    cf1 = coefs[:, :OUT_PAD]
    cf2 = coefs[:, OUT_PAD:2 * OUT_PAD]
    cf3 = coefs[:, 2 * OUT_PAD:]
    padi = jnp.arange(OUT_PAD - n3, dtype=jnp.int32)
    a3p = jnp.concatenate([a3, padi])
    b3p = jnp.concatenate([b3, padi])

    gsz = n3 // NCLS
    g = jnp.arange(OUT_PAD)
    c0w = (g // BPW) * BPW // gsz
    m1 = ((g // gsz == c0w + 1) & (c0w < NCLS - 1)).astype(jnp.float32)
    cf3m = jnp.concatenate(
        [cf3 * (g < n3).astype(jnp.float32)[None, :], m1[None, :]], axis=0)
    srows = [[0.0] * (2 * NW) for _ in range(NCLS)]
    for w in range(NW):
        c0 = w * BPW // gsz
        srows[c0][2 * w] += 1.0
        srows[c0][2 * w + 1] -= 1.0
        if c0 + 1 < NCLS:
            srows[c0 + 1][2 * w + 1] += 1.0
    sel = jnp.asarray(srows, dtype=jnp.float32)

    y1 = _sc_layer(xt, a1, b1, cf1)
    y2 = _sc_layer(y1, a2, b2, cf2)
    partials = _sc_layer3_gsum(y2, a3p, b3p, cf3m)
    cls = _combine_tc(partials, sel)
    return cls.T


# speedup vs baseline: 1.4602x; 1.0223x over previous
import jax
import jax.numpy as jnp
from jax import lax
from jax.experimental import pallas as pl
from jax.experimental.pallas import tpu as pltpu
from jax.experimental.pallas import tpu_sc as plsc

BATCH = 1024
TAU = 30.0
NCLS = 10
NC, NS, L = 2, 16, 16
NW = NC * NS
OUT_PAD = 8192
BPW = OUT_PAD // NW
GRP = 16
NGRP = BPW // GRP
RQ = 8

_GATE_K = (
    (0, 0, 0, 0), (0, 0, 0, 1), (0, 1, 0, -1), (0, 1, 0, 0),
    (0, 0, 1, -1), (0, 0, 1, 0), (0, 1, 1, -2), (0, 1, 1, -1),
    (1, -1, -1, 1), (1, -1, -1, 2), (1, 0, -1, 0), (1, 0, -1, 1),
    (1, -1, 0, 0), (1, -1, 0, 1), (1, 0, 0, -1), (1, 0, 0, 0),
)


def _coef_tc(wall):
    def body(w_ref, k_ref, o_ref):
        w = w_ref[...]
        m = jnp.max(w, axis=-1, keepdims=True)
        e = jnp.exp(w - m)
        p = e / jnp.sum(e, axis=-1, keepdims=True)
        o_ref[...] = jax.lax.dot(p, k_ref[...], precision=lax.Precision.HIGHEST)

    n = wall.shape[0]
    blk = 2048
    return pl.pallas_call(
        body,
        grid=(n // blk,),
        in_specs=[
            pl.BlockSpec((blk, 16), lambda i: (i, 0)),
            pl.BlockSpec((16, 4), lambda i: (0, 0)),
        ],
        out_specs=pl.BlockSpec((blk, 4), lambda i: (i, 0)),
        out_shape=jax.ShapeDtypeStruct((n, 4), jnp.float32),
    )(wall, jnp.asarray(_GATE_K, dtype=jnp.float32))


def _sc_layer(table, cidx, cfs):
    mesh = plsc.VectorSubcoreMesh(core_axis_name="c", subcore_axis_name="s")

    def body(tab, ci, cf, out, civ, cfv,
             cbufs, obufs, sems_c, sems_o):
        wid = lax.axis_index("s") * NC + lax.axis_index("c")
        base = wid * BPW
        pltpu.sync_copy(ci.at[pl.ds(base * 2, 2 * BPW)], civ)
        pltpu.sync_copy(cf.at[:, pl.ds(base, BPW)], cfv)

        def issue(g):
            s = g % 2
            return pltpu.async_copy(
                tab.at[civ.at[pl.ds(g * 2 * GRP, 2 * GRP)]],
                cbufs[s], sems_c[s])

        pend = {0: issue(0)}
        out_pend = {}
        for g in range(NGRP):
            s = g % 2
            if g + 1 < NGRP:
                pend[g + 1] = issue(g + 1)
            pend.pop(g).wait()
            if g >= 2:
                out_pend.pop(g - 2).wait()
            cbuf, obuf = cbufs[s], obufs[s]
            r0 = g * GRP
            c0v = cfv[0, pl.ds(r0, GRP)]
            c1v = cfv[1, pl.ds(r0, GRP)]
            c2v = cfv[2, pl.ds(r0, GRP)]
            c3v = cfv[3, pl.ds(r0, GRP)]
            for q in range(GRP // RQ):
                rows = [q * RQ + i for i in range(RQ)]
                cs = [(c0v[r], c1v[r], c2v[r], c3v[r]) for r in rows]

                def col_fn(j, carry2, rows=rows, cs=cs,
                           cbuf=cbuf, obuf=obuf):
                    sl = pl.ds(j * L, L)
                    for r, (c0, c1, c2, c3) in zip(rows, cs):
                        av = cbuf[r, sl]
                        bv = cbuf[GRP + r, sl]
                        obuf[r, sl] = (c0 + c1 * av) + (c2 + c3 * av) * bv
                    return carry2

                lax.fori_loop(0, BATCH // L, col_fn, 0)
            out_pend[g] = pltpu.async_copy(
                obuf, out.at[pl.ds(base + r0, GRP)], sems_o[s])
        for g in sorted(out_pend):
            out_pend.pop(g).wait()

    kfn = pl.kernel(
        body,
        out_type=jax.ShapeDtypeStruct((OUT_PAD, BATCH), jnp.float32),
        mesh=mesh,
        scratch_types=[
            pltpu.VMEM((2 * BPW,), jnp.int32),
            pltpu.VMEM((4, BPW), jnp.float32),
            [pltpu.VMEM((2 * GRP, BATCH), jnp.float32)] * 2,
            [pltpu.VMEM((GRP, BATCH), jnp.float32)] * 2,
            [pltpu.SemaphoreType.DMA] * 2,
            [pltpu.SemaphoreType.DMA] * 2,
        ],
    )
    return kfn(table, cidx, cfs)


def _sc_layer3_gsum(table, cidx, cfs):
    mesh = plsc.VectorSubcoreMesh(core_axis_name="c", subcore_axis_name="s")

    def body(tab, ci, cf, out, civ, cfv, acc,
             cbufs, sems_c):
        wid = lax.axis_index("s") * NC + lax.axis_index("c")
        base = wid * BPW
        pltpu.sync_copy(ci.at[pl.ds(base * 2, 2 * BPW)], civ)
        pltpu.sync_copy(cf.at[:, pl.ds(base, BPW)], cfv)

        def issue(g):
            s = g % 2
            return pltpu.async_copy(
                tab.at[civ.at[pl.ds(g * 2 * GRP, 2 * GRP)]],
                cbufs[s], sems_c[s])

        pend = {0: issue(0)}
        for g in range(NGRP):
            s = g % 2
            if g + 1 < NGRP:
                pend[g + 1] = issue(g + 1)
            pend.pop(g).wait()
            cbuf = cbufs[s]
            r0 = g * GRP
            c0v = cfv[0, pl.ds(r0, GRP)]
            c1v = cfv[1, pl.ds(r0, GRP)]
            c2v = cfv[2, pl.ds(r0, GRP)]
            c3v = cfv[3, pl.ds(r0, GRP)]
            m1v = cfv[4, pl.ds(r0, GRP)]
            for q in range(GRP // RQ):
                rows = [q * RQ + i for i in range(RQ)]
                cs = [(c0v[r], c1v[r], c2v[r], c3v[r], m1v[r]) for r in rows]
                init = (g == 0 and q == 0)

                def col_fn(j, carry2, rows=rows, cs=cs, init=init,
                           cbuf=cbuf):
                    sl = pl.ds(j * L, L)
                    if init:
                        r0_, (c0, c1, c2, c3, m1) = rows[0], cs[0]
                        av = cbuf[r0_, sl]
                        bv = cbuf[GRP + r0_, sl]
                        val = (c0 + c1 * av) + (c2 + c3 * av) * bv
                        sacc = val
                        tacc = m1 * val
                        rest = list(zip(rows[1:], cs[1:]))
                    else:
                        sacc = acc[0, sl]
                        tacc = acc[1, sl]
                        rest = list(zip(rows, cs))
                    for r, (c0, c1, c2, c3, m1) in rest:
                        av = cbuf[r, sl]
                        bv = cbuf[GRP + r, sl]
                        val = (c0 + c1 * av) + (c2 + c3 * av) * bv
                        sacc = sacc + val
                        tacc = tacc + m1 * val
                    acc[0, sl] = sacc
                    acc[1, sl] = tacc
                    return carry2

                lax.fori_loop(0, BATCH // L, col_fn, 0)
        pltpu.sync_copy(acc, out.at[pl.ds(wid * 2, 2)])

    kfn = pl.kernel(
        body,
        out_type=jax.ShapeDtypeStruct((2 * NW, BATCH), jnp.float32),
        mesh=mesh,
        scratch_types=[
            pltpu.VMEM((2 * BPW,), jnp.int32),
            pltpu.VMEM((5, BPW), jnp.float32),
            pltpu.VMEM((2, BATCH), jnp.float32),
            [pltpu.VMEM((2 * GRP, BATCH), jnp.float32)] * 2,
            [pltpu.SemaphoreType.DMA] * 2,
        ],
    )
    return kfn(table, cidx, cfs)


def _combine_tc(partials, sel):
    def body(s_ref, p_ref, o_ref):
        o_ref[...] = jax.lax.dot(
            s_ref[...], p_ref[...], precision=lax.Precision.HIGHEST) / TAU

    return pl.pallas_call(
        body,
        out_shape=jax.ShapeDtypeStruct((NCLS, BATCH), jnp.float32),
    )(sel, partials)


def kernel(x, w1, w2, w3, a1, b1, a2, b2, a3, b3):
    xt = x.T
    n3 = w3.shape[0]
    w3p = jnp.concatenate([w3, jnp.zeros((OUT_PAD - n3, 16), jnp.float32)], 0)
    wall = jnp.concatenate([w1, w2, w3p], axis=0)
    coefs = _coef_tc(wall).T
    cf1 = coefs[:, :OUT_PAD]
    cf2 = coefs[:, OUT_PAD:2 * OUT_PAD]
    cf3 = coefs[:, 2 * OUT_PAD:]
    padi = jnp.arange(OUT_PAD - n3, dtype=jnp.int32)
    a3p = jnp.concatenate([a3, padi])
    b3p = jnp.concatenate([b3, padi])

    gsz = n3 // NCLS
    g = jnp.arange(OUT_PAD)
    c0w = (g // BPW) * BPW // gsz
    m1 = ((g // gsz == c0w + 1) & (c0w < NCLS - 1)).astype(jnp.float32)
    cf3m = jnp.concatenate(
        [cf3 * (g < n3).astype(jnp.float32)[None, :], m1[None, :]], axis=0)
    srows = [[0.0] * (2 * NW) for _ in range(NCLS)]
    for w in range(NW):
        c0 = w * BPW // gsz
        srows[c0][2 * w] += 1.0
        srows[c0][2 * w + 1] -= 1.0
        if c0 + 1 < NCLS:
            srows[c0 + 1][2 * w + 1] += 1.0
    sel = jnp.asarray(srows, dtype=jnp.float32)

    def interleave(a, b):
        ar = a.reshape(NW, NGRP, GRP)
        br = b.reshape(NW, NGRP, GRP)
        return jnp.concatenate([ar, br], axis=-1).reshape(-1)

    y1 = _sc_layer(xt, interleave(a1, b1), cf1)
    y2 = _sc_layer(y1, interleave(a2, b2), cf2)
    partials = _sc_layer3_gsum(y2, interleave(a3p, b3p), cf3m)
    cls = _combine_tc(partials, sel)
    return cls.T
